# baseline (speedup 1.0000x reference)
import functools

import jax
import jax.numpy as jnp
from jax import lax
from jax.experimental import pallas as pl
from jax.experimental.pallas import tpu as pltpu
from jax.experimental.pallas import tpu_sc as plsc

VOCAB = 1000000
EMBED = 32
BATCH = 16384
HIST = 50

NC = 2
NS = 16
NW = NC * NS
B_TOTAL = BATCH * HIST
PER_W = B_TOTAL // NW
CHUNK = 256
NCH = PER_W // CHUNK
NBUF = 4
NGRP = NCH // NBUF

_mesh = plsc.VectorSubcoreMesh(core_axis_name="c", subcore_axis_name="s")


@functools.partial(
    pl.kernel,
    out_type=jax.ShapeDtypeStruct((NW, NCH, CHUNK, EMBED), jnp.float32),
    mesh=_mesh,
    scratch_types=[
        pltpu.VMEM((NCH, CHUNK), jnp.int32),
        pltpu.VMEM((NBUF, CHUNK, EMBED), jnp.float32),
        pltpu.SemaphoreType.DMA((NBUF,)),
        pltpu.SemaphoreType.DMA((NBUF,)),
    ],
    compiler_params=pltpu.CompilerParams(use_tc_tiling_on_sc=False),
)
def _emb_lookup(idx_hbm, table_hbm, out_hbm, idx_v, rows_v, gsem, wsem):
    wid = lax.axis_index("s") * NC + lax.axis_index("c")
    pltpu.sync_copy(idx_hbm.at[wid], idx_v)

    def g_start(j, b):
        pltpu.async_copy(table_hbm.at[idx_v.at[j]], rows_v.at[b], gsem.at[b])

    def g_wait(b):
        pltpu.make_async_copy(
            table_hbm.at[pl.ds(0, CHUNK)], rows_v.at[b], gsem.at[b]).wait()

    def w_start(j, b):
        pltpu.async_copy(rows_v.at[b], out_hbm.at[wid, j], wsem.at[b])

    def w_wait(b):
        pltpu.make_async_copy(
            rows_v.at[b], out_hbm.at[0, 0], wsem.at[b]).wait()

    for b in range(NBUF):
        g_start(b, b)

    for b in range(NBUF):
        g_wait(b)
        w_start(b, b)
        if b >= 1:
            w_wait(b - 1)
            g_start(b + NBUF - 1, b - 1)

    @pl.loop(1, NGRP - 1)
    def _steady(g):
        j0 = g * NBUF
        for b in range(NBUF):
            g_wait(b)
            w_start(j0 + b, b)
            bp = (b - 1) % NBUF
            w_wait(bp)
            g_start(j0 + b + NBUF - 1, bp)

    j0 = (NGRP - 1) * NBUF
    for b in range(NBUF):
        g_wait(b)
        w_start(j0 + b, b)
        if b == 0:
            w_wait(NBUF - 1)
            g_start(j0 + NBUF - 1, NBUF - 1)

    for b in range(NBUF):
        w_wait(b)


def kernel(indices, table):
    idx = indices.reshape(NW, NCH, CHUNK).astype(jnp.int32)
    out = _emb_lookup(idx, table)
    return out.reshape(BATCH, HIST, EMBED)


# speedup vs baseline: 1.3869x; 1.3869x over previous
import functools

import jax
import jax.numpy as jnp
from jax import lax
from jax.experimental import pallas as pl
from jax.experimental.pallas import tpu as pltpu
from jax.experimental.pallas import tpu_sc as plsc

VOCAB = 1000000
EMBED = 32
BATCH = 16384
HIST = 50

NC = 2
NS = 16
NW = NC * NS
ROWS_W = BATCH // NW
CHUNK = HIST
NCH = ROWS_W
NBUF = 8
NGRP = NCH // NBUF

_mesh = plsc.VectorSubcoreMesh(core_axis_name="c", subcore_axis_name="s")


@functools.partial(
    pl.kernel,
    out_type=jax.ShapeDtypeStruct((BATCH, HIST, EMBED), jnp.float32),
    mesh=_mesh,
    scratch_types=[
        pltpu.VMEM((NCH, CHUNK), jnp.int32),
        pltpu.VMEM((NBUF, CHUNK, EMBED), jnp.float32),
        pltpu.SemaphoreType.DMA((NBUF,)),
        pltpu.SemaphoreType.DMA((NBUF,)),
    ],
    compiler_params=pltpu.CompilerParams(use_tc_tiling_on_sc=False),
)
def _emb_lookup(idx_hbm, table_hbm, out_hbm, idx_v, rows_v, gsem, wsem):
    wid = lax.axis_index("s") * NC + lax.axis_index("c")
    base = wid * ROWS_W
    pltpu.sync_copy(idx_hbm.at[pl.ds(base, ROWS_W)], idx_v)

    def g_start(j, b):
        pltpu.async_copy(table_hbm.at[idx_v.at[j]], rows_v.at[b], gsem.at[b])

    def g_wait(b):
        pltpu.make_async_copy(
            table_hbm.at[pl.ds(0, CHUNK)], rows_v.at[b], gsem.at[b]).wait()

    def w_start(j, b):
        pltpu.async_copy(rows_v.at[b], out_hbm.at[base + j], wsem.at[b])

    def w_wait(b):
        pltpu.make_async_copy(
            rows_v.at[b], out_hbm.at[0], wsem.at[b]).wait()

    for b in range(NBUF):
        g_start(b, b)

    for b in range(NBUF):
        g_wait(b)
        w_start(b, b)
        if b >= 1:
            w_wait(b - 1)
            g_start(b + NBUF - 1, b - 1)

    @pl.loop(1, NGRP - 1)
    def _steady(g):
        j0 = g * NBUF
        for b in range(NBUF):
            g_wait(b)
            w_start(j0 + b, b)
            bp = (b - 1) % NBUF
            w_wait(bp)
            g_start(j0 + b + NBUF - 1, bp)

    j0 = (NGRP - 1) * NBUF
    for b in range(NBUF):
        g_wait(b)
        w_start(j0 + b, b)
        if b == 0:
            w_wait(NBUF - 1)
            g_start(j0 + NBUF - 1, NBUF - 1)

    for b in range(NBUF):
        w_wait(b)


def kernel(indices, table):
    return _emb_lookup(indices.astype(jnp.int32), table)
